# Initial kernel scaffold; baseline (speedup 1.0000x reference)
import jax
import jax.numpy as jnp
from jax.experimental import pallas as pl


def kernel(x, base_weight, ortho_values, ortho_indices, alpha):
    raise NotImplementedError("write your pallas kernel here")


# speedup vs baseline: 1.9515x; 1.9515x over previous
import jax
import jax.numpy as jnp
from jax.experimental import pallas as pl
from jax.experimental.pallas import tpu as pltpu

NNZ = 16384
OUT_F = 1024
IN_F = 1024

NCHUNK = 8
KC = NNZ // NCHUNK
NB = 2
CB = IN_F // NB

BT = 1024


def _scatter_kernel(idx_ref, val_ref, base_ref, alpha_ref, w_ref, acc_ref):
    b = pl.program_id(0)
    k = pl.program_id(1)

    @pl.when(k == 0)
    def _():
        acc_ref[...] = jnp.zeros_like(acc_ref)

    idx = idx_ref[0]
    rows = jax.lax.shift_right_logical(idx, 10)
    cols = jnp.bitwise_and(idx, IN_F - 1) - b * CB
    vals = val_ref[0]

    iota_r = jax.lax.broadcasted_iota(jnp.int32, (OUT_F, KC), 0)
    iota_c = jax.lax.broadcasted_iota(jnp.int32, (CB, KC), 0)

    rt = jnp.where(jnp.broadcast_to(rows, (OUT_F, KC)) == iota_r, 1.0, 0.0
                   ).astype(jnp.bfloat16)
    ct = jnp.where(jnp.broadcast_to(cols, (CB, KC)) == iota_c,
                   jnp.broadcast_to(vals, (CB, KC)), 0.0).astype(jnp.bfloat16)

    acc_ref[...] += jax.lax.dot_general(
        rt, ct, (((1,), (1,)), ((), ())), preferred_element_type=jnp.float32)

    @pl.when(k == NCHUNK - 1)
    def _():
        w_ref[...] = (base_ref[...] + alpha_ref[0, 0] * acc_ref[...]
                      ).astype(jnp.bfloat16)


def _matmul_kernel(x_ref, w_ref, o_ref):
    xb = x_ref[...].astype(jnp.bfloat16)
    o_ref[...] = jax.lax.dot_general(
        xb, w_ref[...], (((1,), (1,)), ((), ())),
        preferred_element_type=jnp.float32)


def _build_w_eff(idx3, vals3, base32, alpha2d, *, interpret=False):
    return pl.pallas_call(
        _scatter_kernel,
        grid=(NB, NCHUNK),
        in_specs=[
            pl.BlockSpec((1, 1, KC), lambda b, k: (k, 0, 0)),
            pl.BlockSpec((1, 1, KC), lambda b, k: (k, 0, 0)),
            pl.BlockSpec((OUT_F, CB), lambda b, k: (0, b)),
            pl.BlockSpec(memory_space=pltpu.SMEM),
        ],
        out_specs=pl.BlockSpec((OUT_F, CB), lambda b, k: (0, b)),
        out_shape=jax.ShapeDtypeStruct((OUT_F, IN_F), jnp.bfloat16),
        scratch_shapes=[pltpu.VMEM((OUT_F, CB), jnp.float32)],
        compiler_params=pltpu.CompilerParams(
            dimension_semantics=("parallel", "arbitrary"),
        ),
        name="ortho_scatter_weff",
        interpret=interpret,
    )(idx3, vals3, base32, alpha2d)


def _apply(xf, w_eff, *, interpret=False):
    t = xf.shape[0]
    return pl.pallas_call(
        _matmul_kernel,
        grid=(t // BT,),
        in_specs=[
            pl.BlockSpec((BT, IN_F), lambda i: (i, 0)),
            pl.BlockSpec((OUT_F, IN_F), lambda i: (0, 0)),
        ],
        out_specs=pl.BlockSpec((BT, OUT_F), lambda i: (i, 0)),
        out_shape=jax.ShapeDtypeStruct((t, OUT_F), jnp.float32),
        compiler_params=pltpu.CompilerParams(
            dimension_semantics=("parallel",),
        ),
        name="ortho_linear_matmul",
        interpret=interpret,
    )(xf, w_eff)


def kernel(x, base_weight, ortho_values, ortho_indices, alpha, *, interpret=False):
    out_f, in_f = base_weight.shape
    lead = x.shape[:-1]
    xf = x.reshape(-1, in_f)

    idx3 = ortho_indices.reshape(NCHUNK, 1, KC)
    vals3 = ortho_values.astype(jnp.float32).reshape(NCHUNK, 1, KC)
    base32 = base_weight.astype(jnp.float32)
    alpha2d = alpha.astype(jnp.float32).reshape(1, 1)

    w_eff = _build_w_eff(idx3, vals3, base32, alpha2d, interpret=interpret)
    out = _apply(xf, w_eff, interpret=interpret)
    return out.reshape(*lead, out_f)
